# Initial kernel scaffold; baseline (speedup 1.0000x reference)
import jax
import jax.numpy as jnp
from jax.experimental import pallas as pl


def kernel(x, edge_index, edge_type, W1, W1_comp, W2, W2_comp):
    raise NotImplementedError("write your pallas kernel here")


# speedup vs baseline: 4.9561x; 4.9561x over previous
import functools

import jax
import jax.numpy as jnp
from jax import lax
from jax.experimental import pallas as pl
from jax.experimental.pallas import tpu as pltpu
from jax.experimental.pallas import tpu_sc as plsc

N = 10000
E = 160000
D_IN = 256
D_HID = 256
D_OUT = 16
NB = 4
NS = 4
NT = 16
NC = 2
LANES = 16

TN = 2000
GRID = N // TN

NPAD = 10240
EPT1 = E // NT
K1 = 80
NCH1 = EPT1 // K1
RPT = NPAD // NT
ZR1 = 128

EPC = E // NC
EPT2 = EPC // NT
K2 = 80
NCH2 = (EPT2 + K2 - 1) // K2
SLOTS2 = NCH2 * K2


def _l1_body(x_ref, w_ref, comp_ref, out_ref):
    x = x_ref[...]
    w = w_ref[...]
    comp = comp_ref[...]
    xb = [jnp.dot(x, w[b * D_IN:(b + 1) * D_IN, :],
                  preferred_element_type=jnp.float32) for b in range(NB)]
    for r in range(NS):
        acc = comp[r:r + 1, 0:1] * xb[0]
        for b in range(1, NB):
            acc = acc + comp[r:r + 1, b:b + 1] * xb[b]
        out_ref[r] = acc


def _l1_matmul(x, W1, W1_comp):
    return pl.pallas_call(
        _l1_body,
        grid=(GRID,),
        in_specs=[
            pl.BlockSpec((TN, D_IN), lambda i: (i, 0)),
            pl.BlockSpec((NB * D_IN, D_HID), lambda i: (0, 0)),
            pl.BlockSpec((NS, NB), lambda i: (0, 0)),
        ],
        out_specs=pl.BlockSpec((NS, TN, D_HID), lambda i: (0, i, 0)),
        out_shape=jax.ShapeDtypeStruct((NS, N, D_HID), jnp.float32),
    )(x, W1, W1_comp)


def _l2_body(p_ref, w_ref, comp_ref, out_ref):
    h = jnp.maximum(
        jnp.concatenate([p_ref[q] for q in range(4)], axis=1), 0.0)
    w = w_ref[...]
    comp = comp_ref[...]
    wcat = jnp.concatenate(
        [w[b * D_HID:(b + 1) * D_HID, :] for b in range(NB)], axis=1)
    hb = jnp.dot(h, wcat, preferred_element_type=jnp.float32)
    for r in range(NS):
        acc = comp[r:r + 1, 0:1] * hb[:, 0:D_OUT]
        for b in range(1, NB):
            acc = acc + comp[r:r + 1, b:b + 1] * hb[:, b * D_OUT:(b + 1) * D_OUT]
        out_ref[r] = acc


def _l2_matmul(parts, W2, W2_comp):
    return pl.pallas_call(
        _l2_body,
        grid=(GRID,),
        in_specs=[
            pl.BlockSpec((4, TN, 64), lambda i: (0, i, 0)),
            pl.BlockSpec((NB * D_HID, D_OUT), lambda i: (0, 0)),
            pl.BlockSpec((NS, NB), lambda i: (0, 0)),
        ],
        out_specs=pl.BlockSpec((NS, TN, D_OUT), lambda i: (0, i, 0)),
        out_shape=jax.ShapeDtypeStruct((NS, N, D_OUT), jnp.float32),
    )(parts, W2, W2_comp)


def _softmax_body(p_ref, out_ref):
    s = p_ref[0] + p_ref[1]
    m = jnp.max(s, axis=1, keepdims=True)
    e = jnp.exp(s - m)
    out_ref[...] = e / jnp.sum(e, axis=1, keepdims=True)


def _softmax_sum(parts):
    return pl.pallas_call(
        _softmax_body,
        grid=(GRID,),
        in_specs=[pl.BlockSpec((NC, TN, D_OUT), lambda i: (0, i, 0))],
        out_specs=pl.BlockSpec((TN, D_OUT), lambda i: (i, 0)),
        out_shape=jax.ShapeDtypeStruct((N, D_OUT), jnp.float32),
    )(parts)


_MESH = plsc.VectorSubcoreMesh(core_axis_name="c", subcore_axis_name="s")


@functools.partial(
    pl.kernel,
    out_type=jax.ShapeDtypeStruct((4, NPAD, 64), jnp.float32),
    mesh=_MESH,
    compiler_params=pltpu.CompilerParams(use_tc_tiling_on_sc=False),
    scratch_types=[
        pltpu.VMEM((EPT1,), jnp.int32),
        pltpu.VMEM((EPT1,), jnp.int32),
        pltpu.VMEM((EPT1,), jnp.int32),
        pltpu.VMEM((NCH1, K1), jnp.int32),
        pltpu.VMEM((NCH1, K1), jnp.int32),
        pltpu.VMEM((K1, 64), jnp.float32),
        pltpu.VMEM((ZR1, 64), jnp.float32),
        pltpu.VMEM_SHARED((NPAD, 64), jnp.float32),
        pltpu.SemaphoreType.DMA,
    ],
)
def _agg1(y1_hbm, src_hbm, dst_hbm, type_hbm, out_hbm,
          srcv, dstv, typev, gidx, didx, rows, zbuf, acc, sem):
    c = lax.axis_index("c")
    s = lax.axis_index("s")
    ebase = s * EPT1

    pltpu.sync_copy(src_hbm.at[pl.ds(ebase, EPT1)], srcv)
    pltpu.sync_copy(dst_hbm.at[pl.ds(ebase, EPT1)], dstv)
    pltpu.sync_copy(type_hbm.at[pl.ds(ebase, EPT1)], typev)

    zv = jnp.zeros((LANES,), jnp.float32)

    def zfill(i, carry):
        for k in range(64 // LANES):
            zbuf[i, pl.ds(k * LANES, LANES)] = zv
        return carry

    def dfill(j, carry):
        for k in range(K1 // LANES):
            off = j * K1 + k * LANES
            didx[j, pl.ds(k * LANES, LANES)] = dstv[pl.ds(off, LANES)]
        return carry

    lax.fori_loop(0, NCH1, dfill, 0)

    r0 = s * RPT
    for p in range(2):
        q = 2 * c + p

        def ifill(j, carry):
            for k in range(K1 // LANES):
                off = j * K1 + k * LANES
                sv = srcv[pl.ds(off, LANES)]
                tv = typev[pl.ds(off, LANES)]
                gidx[j, pl.ds(k * LANES, LANES)] = (tv * N + sv) * 4 + q
            return carry

        lax.fori_loop(0, NCH1, ifill, 0)
        lax.fori_loop(0, ZR1, zfill, 0)

        for z in range(RPT // ZR1):
            pltpu.sync_copy(zbuf, acc.at[pl.ds(r0 + z * ZR1, ZR1)])

        plsc.subcore_barrier()

        def chunk(j, carry):
            pltpu.async_copy(y1_hbm.at[gidx.at[j]], rows, sem).wait()
            pltpu.sync_copy(rows, acc.at[didx.at[j]], add=True)
            return carry

        lax.fori_loop(0, NCH1, chunk, 0)

        plsc.subcore_barrier()

        for z in range(RPT // ZR1):
            pltpu.sync_copy(acc.at[pl.ds(r0 + z * ZR1, ZR1)], zbuf)
            pltpu.sync_copy(zbuf, out_hbm.at[q, pl.ds(r0 + z * ZR1, ZR1)])


@functools.partial(
    pl.kernel,
    out_type=jax.ShapeDtypeStruct((NC, NPAD, D_OUT), jnp.float32),
    mesh=_MESH,
    compiler_params=pltpu.CompilerParams(use_tc_tiling_on_sc=False),
    scratch_types=[
        pltpu.VMEM((SLOTS2,), jnp.int32),
        pltpu.VMEM((SLOTS2,), jnp.int32),
        pltpu.VMEM((SLOTS2,), jnp.int32),
        pltpu.VMEM((NCH2, K2), jnp.int32),
        pltpu.VMEM((NCH2, K2), jnp.int32),
        pltpu.VMEM((K2, D_OUT), jnp.float32),
        pltpu.VMEM((RPT, D_OUT), jnp.float32),
        pltpu.VMEM_SHARED((NPAD, D_OUT), jnp.float32),
        pltpu.SemaphoreType.DMA,
    ],
)
def _agg2(y2_hbm, src_hbm, dst_hbm, type_hbm, out_hbm,
          srcv, dstv, typev, gidx, didx, rows, zbuf, acc, sem):
    c = lax.axis_index("c")
    s = lax.axis_index("s")
    ebase = c * EPC + s * EPT2

    pltpu.sync_copy(src_hbm.at[pl.ds(ebase, EPT2)], srcv.at[pl.ds(0, EPT2)])
    pltpu.sync_copy(dst_hbm.at[pl.ds(ebase, EPT2)], dstv.at[pl.ds(0, EPT2)])
    pltpu.sync_copy(type_hbm.at[pl.ds(ebase, EPT2)], typev.at[pl.ds(0, EPT2)])

    zv = jnp.zeros((LANES,), jnp.float32)

    def zfill(i, carry):
        zbuf[i] = zv
        return carry

    lax.fori_loop(0, RPT, zfill, 0)
    pltpu.sync_copy(zbuf, acc.at[pl.ds(s * RPT, RPT)])

    lane = lax.broadcasted_iota(jnp.int32, (LANES,), 0)

    def ifill(j, carry):
        for k in range(K2 // LANES):
            off = j * K2 + k * LANES
            sv = srcv[pl.ds(off, LANES)]
            tv = typev[pl.ds(off, LANES)]
            dv = dstv[pl.ds(off, LANES)]
            valid = (off + lane) < EPT2
            gidx[j, pl.ds(k * LANES, LANES)] = jnp.where(valid, tv * N + sv, 0)
            didx[j, pl.ds(k * LANES, LANES)] = jnp.where(valid, dv, N)
        return carry

    lax.fori_loop(0, NCH2, ifill, 0)

    plsc.subcore_barrier()

    def chunk(j, carry):
        pltpu.async_copy(y2_hbm.at[gidx.at[j]], rows, sem).wait()
        pltpu.sync_copy(rows, acc.at[didx.at[j]], add=True)
        return carry

    lax.fori_loop(0, NCH2, chunk, 0)

    plsc.subcore_barrier()

    pltpu.sync_copy(acc.at[pl.ds(s * RPT, RPT)], zbuf)
    pltpu.sync_copy(zbuf, out_hbm.at[c, pl.ds(s * RPT, RPT)])


def kernel(x, edge_index, edge_type, W1, W1_comp, W2, W2_comp):
    src = edge_index[0]
    dst = edge_index[1]
    y1 = _l1_matmul(x, W1, W1_comp)
    y1s = y1.reshape(NS * N * 4, 64)
    h1p = _agg1(y1s, src, dst, edge_type)
    y2 = _l2_matmul(h1p, W2, W2_comp)
    y2s = y2.reshape(NS * N, D_OUT)
    parts = _agg2(y2s, src, dst, edge_type)
    return _softmax_sum(parts)
